# Initial kernel scaffold; baseline (speedup 1.0000x reference)
import jax
import jax.numpy as jnp
from jax.experimental import pallas as pl


def kernel(img):
    raise NotImplementedError("write your pallas kernel here")


# speedup vs baseline: 1.0074x; 1.0074x over previous
import jax
import jax.numpy as jnp
from jax.experimental import pallas as pl


def _copy_body(img_ref, out_ref):
    out_ref[...] = img_ref[...]


def kernel(img):
    return pl.pallas_call(
        _copy_body,
        out_shape=jax.ShapeDtypeStruct(img.shape, img.dtype),
    )(img)
